# baseline (speedup 1.0000x reference)
import functools

import jax
import jax.numpy as jnp
from jax import lax
from jax.experimental import pallas as pl
from jax.experimental.pallas import tpu as pltpu
from jax.experimental.pallas import tpu_sc as plsc

_VOCAB = 100000
_D = 64
_B = 4096
_S = 256

_NC = 2
_NS = 16
_NW = _NC * _NS

_TOTAL = _B * _S
_PER_W = _TOTAL // _NW
_C = 128
_NCHUNK = _PER_W // _C
_NE = 4
_NT = 4
_NI = 8


def _make_kernel():
    mesh = plsc.VectorSubcoreMesh(core_axis_name="c", subcore_axis_name="s")

    @functools.partial(
        pl.kernel,
        mesh=mesh,
        out_type=jax.ShapeDtypeStruct((_B * 128, _C), jnp.float32),
        scratch_types=[
            pltpu.VMEM((_NCHUNK, _C), jnp.int32),
            pltpu.VMEM((_NI, _C), jnp.int32),
            pltpu.VMEM((_NE, _C, _D), jnp.float32),
            pltpu.VMEM((_NT, _D, 129), jnp.float32),
            pltpu.SemaphoreType.DMA((_NI,)),
            pltpu.SemaphoreType.DMA((_NE,)),
            pltpu.SemaphoreType.DMA((_NT,)),
        ],
        compiler_params=pltpu.CompilerParams(use_tc_tiling_on_sc=False,
                                             needs_layout_passes=False),
    )
    def k(tok_hbm, vmap_hbm, table_hbm, out_hbm,
          tok_all, ids, emb, embt, isem, gsem, ssem):
        wid = lax.axis_index("s") * _NC + lax.axis_index("c")
        brow = wid * _NCHUNK
        lane = lax.iota(jnp.int32, 16)

        pltpu.sync_copy(tok_hbm.at[pl.ds(brow, _NCHUNK)], tok_all)

        def fire_ids(g, slot):
            pltpu.async_copy(vmap_hbm.at[tok_all.at[g]], ids.at[slot],
                             isem.at[slot])

        def wait_ids(slot):
            pltpu.make_async_copy(vmap_hbm.at[pl.ds(0, _C)], ids.at[slot],
                                  isem.at[slot]).wait()

        def fire_emb(g, eslot, islot):
            del g
            pltpu.async_copy(table_hbm.at[ids.at[islot]], emb.at[eslot],
                             gsem.at[eslot])

        def wait_emb(eslot):
            pltpu.make_async_copy(table_hbm.at[pl.ds(0, _C)], emb.at[eslot],
                                  gsem.at[eslot]).wait()

        def fire_out(g, tslot, eslot):
            del eslot
            row0 = wid * (128 * 128) + (g // 2) * 128 + (g % 2) * 8
            for i in range(8):
                pltpu.async_copy(embt.at[tslot, pl.ds(i * 8, 8), pl.ds(0, _C)],
                                 out_hbm.at[pl.ds(row0 + i * 16, 8)],
                                 ssem.at[tslot])

        def wait_out(tslot):
            pltpu.make_async_copy(table_hbm.at[pl.ds(0, _C)],
                                  emb.at[0].at[pl.ds(0, _C)],
                                  ssem.at[tslot]).wait()

        tvecs = [jnp.full((16,), t, jnp.int32) for t in range(_NT)]
        dvecs = [d0 + lane for d0 in range(0, _D, 16)]

        def transpose(eslot, tslot):
            @plsc.parallel_loop(0, _C, unroll=8)
            def sbody(s):
                srow = jnp.full((16,), 0, jnp.int32) + s
                for q in range(4):
                    v = emb[eslot, s, pl.ds(q * 16, 16)]
                    plsc.store_scatter(embt, [tvecs[tslot], dvecs[q], srow], v)

        def step(g, j, *, do_fire_emb, do_fire_ids, do_wait_out):
            if do_fire_emb:
                wait_ids((j + 2) % _NI)
                fire_emb(g + 2, (j + 2) % _NE, (j + 2) % _NI)
            wait_emb(j % _NE)
            if do_fire_ids:
                fire_ids(g + _NI, j % _NI)
            if do_wait_out:
                wait_out(j % _NT)
            transpose(j % _NE, j % _NT)
            fire_out(g, j % _NT, j % _NE)

        for c in range(_NI):
            fire_ids(c, c)
        wait_ids(0)
        fire_emb(0, 0, 0)
        wait_ids(1)
        fire_emb(1, 1, 1)

        for j in range(_NI):
            step(j, j, do_fire_emb=True, do_fire_ids=True,
                 do_wait_out=(j >= _NT))

        def super_body(s, carry):
            g0 = s * _NI
            for j in range(_NI):
                step(g0 + j, j, do_fire_emb=True, do_fire_ids=True,
                     do_wait_out=True)
            return carry
        lax.fori_loop(1, _NCHUNK // _NI - 1, super_body, 0, unroll=False)

        for j in range(_NI):
            g = _NCHUNK - _NI + j
            step(g, j, do_fire_emb=(g + 2 < _NCHUNK), do_fire_ids=False,
                 do_wait_out=True)

        for t in range(_NT):
            wait_out(t)

    return k


_k = _make_kernel()


def kernel(tokens, vocab_map, embed_table):
    b = tokens.shape[0]
    vmap_ext = jnp.concatenate(
        [vocab_map, jnp.array([0, 2], dtype=vocab_map.dtype)])
    bos = jnp.full((b, 1), _VOCAB, dtype=tokens.dtype)
    eos = jnp.full((b, 1), _VOCAB + 1, dtype=tokens.dtype)
    tok_ext = jnp.concatenate([bos, tokens, eos], axis=1).reshape(-1, _C)
    out = _k(tok_ext, vmap_ext, embed_table)
    return out.reshape(b, 8, 2, 8, _C).transpose(0, 2, 4, 1, 3).reshape(
        b, _S, _D)


# speedup vs baseline: 1.0186x; 1.0186x over previous
import functools

import jax
import jax.numpy as jnp
from jax import lax
from jax.experimental import pallas as pl
from jax.experimental.pallas import tpu as pltpu
from jax.experimental.pallas import tpu_sc as plsc

_VOCAB = 100000
_D = 64
_B = 4096
_S = 256

_NC = 2
_NS = 16
_NW = _NC * _NS

_TOTAL = _B * _S
_PER_W = _TOTAL // _NW
_C = 128
_NCHUNK = _PER_W // _C
_NE = 4
_NT = 4
_NI = 8


def _make_kernel():
    mesh = plsc.VectorSubcoreMesh(core_axis_name="c", subcore_axis_name="s")

    @functools.partial(
        pl.kernel,
        mesh=mesh,
        out_type=jax.ShapeDtypeStruct((_B * 128, _C), jnp.float32),
        scratch_types=[
            pltpu.VMEM((_NCHUNK, _C), jnp.int32),
            pltpu.VMEM((_NI, _C), jnp.int32),
            pltpu.VMEM((_NE, _C, _D), jnp.float32),
            pltpu.VMEM((_NT, _D, 129), jnp.float32),
            pltpu.VMEM_SHARED((_VOCAB + 2,), jnp.int32),
            pltpu.SemaphoreType.DMA((_NI,)),
            pltpu.SemaphoreType.DMA((_NE,)),
            pltpu.SemaphoreType.DMA((_NT,)),
        ],
        compiler_params=pltpu.CompilerParams(use_tc_tiling_on_sc=False,
                                             needs_layout_passes=False),
    )
    def k(tok_hbm, vmap_hbm, table_hbm, out_hbm,
          tok_all, ids, emb, embt, vmap_sh, isem, gsem, ssem):
        wid = lax.axis_index("s") * _NC + lax.axis_index("c")
        brow = wid * _NCHUNK
        lane = lax.iota(jnp.int32, 16)

        @pl.when(lax.axis_index("s") == 0)
        def _():
            pltpu.sync_copy(vmap_hbm, vmap_sh)
        plsc.subcore_barrier()

        pltpu.sync_copy(tok_hbm.at[pl.ds(brow, _NCHUNK)], tok_all)

        def fire_ids(g, slot):
            pltpu.async_copy(vmap_sh.at[tok_all.at[g]], ids.at[slot],
                             isem.at[slot])

        def wait_ids(slot):
            pltpu.make_async_copy(vmap_hbm.at[pl.ds(0, _C)], ids.at[slot],
                                  isem.at[slot]).wait()

        def fire_emb(g, eslot, islot):
            del g
            pltpu.async_copy(table_hbm.at[ids.at[islot]], emb.at[eslot],
                             gsem.at[eslot])

        def wait_emb(eslot):
            pltpu.make_async_copy(table_hbm.at[pl.ds(0, _C)], emb.at[eslot],
                                  gsem.at[eslot]).wait()

        def fire_out(g, tslot, eslot):
            del eslot
            row0 = wid * (128 * 128) + (g // 2) * 128 + (g % 2) * 8
            for i in range(8):
                pltpu.async_copy(embt.at[tslot, pl.ds(i * 8, 8), pl.ds(0, _C)],
                                 out_hbm.at[pl.ds(row0 + i * 16, 8)],
                                 ssem.at[tslot])

        def wait_out(tslot):
            pltpu.make_async_copy(table_hbm.at[pl.ds(0, _C)],
                                  emb.at[0].at[pl.ds(0, _C)],
                                  ssem.at[tslot]).wait()

        tvecs = [jnp.full((16,), t, jnp.int32) for t in range(_NT)]
        dvecs = [d0 + lane for d0 in range(0, _D, 16)]

        def transpose(eslot, tslot):
            @plsc.parallel_loop(0, _C, unroll=4)
            def sbody(s):
                srow = jnp.full((16,), 0, jnp.int32) + s
                for q in range(4):
                    v = emb[eslot, s, pl.ds(q * 16, 16)]
                    plsc.store_scatter(embt, [tvecs[tslot], dvecs[q], srow], v)

        def step(g, j, *, do_fire_emb, do_fire_ids, do_wait_out):
            if do_fire_emb:
                wait_ids((j + 2) % _NI)
                fire_emb(g + 2, (j + 2) % _NE, (j + 2) % _NI)
            wait_emb(j % _NE)
            if do_fire_ids:
                fire_ids(g + _NI, j % _NI)
            if do_wait_out:
                wait_out(j % _NT)
            transpose(j % _NE, j % _NT)
            fire_out(g, j % _NT, j % _NE)

        for c in range(_NI):
            fire_ids(c, c)
        wait_ids(0)
        fire_emb(0, 0, 0)
        wait_ids(1)
        fire_emb(1, 1, 1)

        for j in range(_NI):
            step(j, j, do_fire_emb=True, do_fire_ids=True,
                 do_wait_out=(j >= _NT))

        def super_body(s, carry):
            g0 = s * _NI
            for j in range(_NI):
                step(g0 + j, j, do_fire_emb=True, do_fire_ids=True,
                     do_wait_out=True)
            return carry
        lax.fori_loop(1, _NCHUNK // _NI - 1, super_body, 0, unroll=False)

        for j in range(_NI):
            g = _NCHUNK - _NI + j
            step(g, j, do_fire_emb=(g + 2 < _NCHUNK), do_fire_ids=False,
                 do_wait_out=True)

        for t in range(_NT):
            wait_out(t)

    return k


_k = _make_kernel()


def kernel(tokens, vocab_map, embed_table):
    b = tokens.shape[0]
    vmap_ext = jnp.concatenate(
        [vocab_map, jnp.array([0, 2], dtype=vocab_map.dtype)])
    bos = jnp.full((b, 1), _VOCAB, dtype=tokens.dtype)
    eos = jnp.full((b, 1), _VOCAB + 1, dtype=tokens.dtype)
    tok_ext = jnp.concatenate([bos, tokens, eos], axis=1).reshape(-1, _C)
    out = _k(tok_ext, vmap_ext, embed_table)
    return out.reshape(b, 8, 2, 8, _C).transpose(0, 2, 4, 1, 3).reshape(
        b, _S, _D)
